# baseline (speedup 1.0000x reference)
import functools

import jax
import jax.numpy as jnp
from jax import lax
from jax.experimental import pallas as pl
from jax.experimental.pallas import tpu as pltpu
from jax.experimental.pallas import tpu_sc as plsc

N = 10000
E = 320000
D = 128
L = 5
BN_EPS = 1e-5
BN_INV = 1.0 / (1.0 + BN_EPS) ** 0.5

NC = 2
NS = 16
NW = NC * NS
EPT = E // NW
CHUNK = 40
NCHUNK = EPT // CHUNK
NIB = 8
NRB = 4
SE = 8 + 8 * ((NCHUNK - 10) // 8)
RPT = 624
RTAIL = N - NS * RPT

BLK = 2000


def _agg_body(h_hbm, src_hbm, dst_hbm, zero_hbm, out_hbm,
              sb0, sb1, sb2, sb3, sb4, sb5, sb6, sb7,
              db0, db1, db2, db3, db4, db5, db6, db7,
              r0, r1, r2, r3, acc_sh, isem, gsem, ssem, zsem):
    c = lax.axis_index("c")
    s = lax.axis_index("s")
    wid = c * NS + s
    base = wid * EPT
    row0 = s * RPT
    sb = [sb0, sb1, sb2, sb3, sb4, sb5, sb6, sb7]
    db = [db0, db1, db2, db3, db4, db5, db6, db7]
    rr = [r0, r1, r2, r3]

    zmain = pltpu.make_async_copy(zero_hbm.at[pl.ds(row0, RPT)],
                                  acc_sh.at[pl.ds(row0, RPT)], zsem)
    ztail = pltpu.make_async_copy(zero_hbm.at[pl.ds(NS * RPT, RTAIL)],
                                  acc_sh.at[pl.ds(NS * RPT, RTAIL)], zsem)
    zmain.start()

    @pl.when(s == NS - 1)
    def _():
        ztail.start()

    def idx_start(k, j):
        off = base + k * CHUNK
        pltpu.make_async_copy(src_hbm.at[pl.ds(off, CHUNK)], sb[j], isem).start()
        pltpu.make_async_copy(dst_hbm.at[pl.ds(off, CHUNK)], db[j], isem).start()

    def idx_wait(j):
        pltpu.make_async_copy(src_hbm.at[pl.ds(0, CHUNK)], sb[j], isem).wait()
        pltpu.make_async_copy(dst_hbm.at[pl.ds(0, CHUNK)], db[j], isem).wait()

    def g_start(j, b):
        pltpu.make_async_copy(h_hbm.at[sb[j]], rr[b], gsem).start()

    def g_wait(j, b):
        pltpu.make_async_copy(h_hbm.at[sb[j]], rr[b], gsem).wait()

    def s_start(j, b):
        pltpu.make_async_copy(rr[b], acc_sh.at[db[j]], ssem).start(add=True)

    def s_wait(j, b):
        pltpu.make_async_copy(rr[b], acc_sh.at[db[j]], ssem).wait()


    for q in range(6):
        idx_start(q, q)
    for q in range(2):
        idx_wait(q)
        g_start(q, q % NRB)

    zmain.wait()

    @pl.when(s == NS - 1)
    def _():
        ztail.wait()

    plsc.subcore_barrier()
    for q in range(8):
        g_wait(q % NIB, q % NRB)
        s_start(q % NIB, q % NRB)
        if q >= 2:
            s_wait((q - 2) % NIB, (q - 2) % NRB)
        idx_start(q + 6, (q + 6) % NIB)
        idx_wait((q + 2) % NIB)
        g_start((q + 2) % NIB, (q + 2) % NRB)

    @pl.loop(8, SE, step=8)
    def _(k):
        for p in range(8):
            g_wait(p, p % NRB)
            s_start(p, p % NRB)
            s_wait((p - 2) % NIB, (p - 2) % NRB)

            @pl.when(k + p + 6 < NCHUNK)
            def _(p=p):
                idx_start(k + p + 6, (p + 6) % NIB)

            idx_wait((p + 2) % NIB)
            g_start((p + 2) % NIB, (p + 2) % NRB)

    for q in range(SE, NCHUNK):
        g_wait(q % NIB, q % NRB)
        s_start(q % NIB, q % NRB)
        s_wait((q - 2) % NIB, (q - 2) % NRB)
        if q + 2 < NCHUNK:
            idx_wait((q + 2) % NIB)
            g_start((q + 2) % NIB, (q + 2) % NRB)
    s_wait((NCHUNK - 2) % NIB, (NCHUNK - 2) % NRB)
    s_wait((NCHUNK - 1) % NIB, (NCHUNK - 1) % NRB)

    plsc.subcore_barrier()
    pltpu.sync_copy(acc_sh.at[pl.ds(row0, RPT)],
                    out_hbm.at[pl.ds(c * N + row0, RPT)])

    @pl.when(s == NS - 1)
    def _():
        pltpu.sync_copy(acc_sh.at[pl.ds(NS * RPT, RTAIL)],
                        out_hbm.at[pl.ds(c * N + NS * RPT, RTAIL)])


@functools.lru_cache(maxsize=1)
def _make_sc_aggregate():
    mesh = plsc.VectorSubcoreMesh(core_axis_name="c", subcore_axis_name="s")
    return pl.kernel(
        _agg_body,
            out_type=jax.ShapeDtypeStruct((NC * N, D), jnp.float32),
        mesh=mesh,
        scratch_types=(
            [pltpu.VMEM((CHUNK,), jnp.int32)] * 16
            + [pltpu.VMEM((CHUNK, D), jnp.float32)] * 4
            + [pltpu.VMEM_SHARED((N, D), jnp.float32),
               pltpu.SemaphoreType.DMA,
               pltpu.SemaphoreType.DMA,
               pltpu.SemaphoreType.DMA,
               pltpu.SemaphoreType.DMA]
        ),
    )


def _mlp_body(eps_ref, h_ref, a0_ref, a1_ref, w1_ref, b1_ref, g1_ref, be1_ref,
              w2_ref, b2_ref, g2_ref, be2_ref, o_ref):
    z = h_ref[...] * (1.0 + eps_ref[0, 0]) + a0_ref[...] + a1_ref[...]
    z = jnp.dot(z, w1_ref[...], preferred_element_type=jnp.float32,
                precision=lax.Precision.HIGHEST)
    z = jnp.maximum(z + b1_ref[...], 0.0)
    z = z * (g1_ref[...] * BN_INV) + be1_ref[...]
    z = jnp.dot(z, w2_ref[...], preferred_element_type=jnp.float32,
                precision=lax.Precision.HIGHEST)
    z = (z + b2_ref[...]) * (g2_ref[...] * BN_INV) + be2_ref[...]
    o_ref[...] = jnp.maximum(z, 0.0)


def _mlp_call(h, agg, eps, w1, b1, g1, be1, w2, b2, g2, be2):
    eps2 = eps.reshape(1, 1)
    row = lambda v: v.reshape(1, D)
    vec_spec = pl.BlockSpec((1, D), lambda i: (0, 0))
    mat_spec = pl.BlockSpec((D, D), lambda i: (0, 0))
    blk_spec = pl.BlockSpec((BLK, D), lambda i: (i, 0))
    return pl.pallas_call(
        _mlp_body,
        grid=(N // BLK,),
        in_specs=[
            pl.BlockSpec(memory_space=pltpu.SMEM),
            blk_spec,
            blk_spec,
            pl.BlockSpec((BLK, D), lambda i: (i + N // BLK, 0)),
            mat_spec, vec_spec, vec_spec, vec_spec,
            mat_spec, vec_spec, vec_spec, vec_spec,
        ],
        out_specs=blk_spec,
        out_shape=jax.ShapeDtypeStruct((N, D), jnp.float32),
    )(eps2, h, agg, agg, w1, row(b1), row(g1), row(be1),
      w2, row(b2), row(g2), row(be2))


def kernel(x, edge_index, batch, params):
    del batch
    src = edge_index[0]
    dst = edge_index[1]
    zeros = jnp.zeros((N, D), jnp.float32)
    sc_aggregate = _make_sc_aggregate()
    h = x
    for i in range(L):
        agg = sc_aggregate(h, src, dst, zeros)
        h = _mlp_call(h, agg, params[f"eps{i}"],
                      params[f"W1_{i}"], params[f"b1_{i}"],
                      params[f"g1_{i}"], params[f"be1_{i}"],
                      params[f"W2_{i}"], params[f"b2_{i}"],
                      params[f"g_{i}"], params[f"be_{i}"])
    return h


# speedup vs baseline: 1.2262x; 1.2262x over previous
import functools

import jax
import jax.numpy as jnp
from jax import lax
from jax.experimental import pallas as pl
from jax.experimental.pallas import tpu as pltpu
from jax.experimental.pallas import tpu_sc as plsc

N = 10000
E = 320000
D = 128
L = 5
BN_EPS = 1e-5
BN_INV = 1.0 / (1.0 + BN_EPS) ** 0.5

NC = 2
NS = 16
NW = NC * NS
EPT = E // NW
CHUNK = 80
NCHUNK = EPT // CHUNK
NIB = 8
NRB = 4
SE = 8 + 8 * ((NCHUNK - 10) // 8)
RPT = 624
RTAIL = N - NS * RPT

BLK = 2000


def _agg_body(h_hbm, src_hbm, dst_hbm, zero_hbm, out_hbm,
              sb0, sb1, sb2, sb3, sb4, sb5, sb6, sb7,
              db0, db1, db2, db3, db4, db5, db6, db7,
              r0, r1, r2, r3, acc_sh, isem, gsem, ssem, zsem):
    c = lax.axis_index("c")
    s = lax.axis_index("s")
    wid = c * NS + s
    base = wid * EPT
    row0 = s * RPT
    sb = [sb0, sb1, sb2, sb3, sb4, sb5, sb6, sb7]
    db = [db0, db1, db2, db3, db4, db5, db6, db7]
    rr = [r0, r1, r2, r3]

    zmain = pltpu.make_async_copy(zero_hbm.at[pl.ds(row0, RPT)],
                                  acc_sh.at[pl.ds(row0, RPT)], zsem)
    ztail = pltpu.make_async_copy(zero_hbm.at[pl.ds(NS * RPT, RTAIL)],
                                  acc_sh.at[pl.ds(NS * RPT, RTAIL)], zsem)
    zmain.start()

    @pl.when(s == NS - 1)
    def _():
        ztail.start()

    def idx_start(k, j):
        off = base + k * CHUNK
        pltpu.make_async_copy(src_hbm.at[pl.ds(off, CHUNK)], sb[j], isem).start()
        pltpu.make_async_copy(dst_hbm.at[pl.ds(off, CHUNK)], db[j], isem).start()

    def idx_wait(j):
        pltpu.make_async_copy(src_hbm.at[pl.ds(0, CHUNK)], sb[j], isem).wait()
        pltpu.make_async_copy(dst_hbm.at[pl.ds(0, CHUNK)], db[j], isem).wait()

    def g_start(j, b):
        pltpu.make_async_copy(h_hbm.at[sb[j]], rr[b], gsem).start()

    def g_wait(j, b):
        pltpu.make_async_copy(h_hbm.at[sb[j]], rr[b], gsem).wait()

    def s_start(j, b):
        pltpu.make_async_copy(rr[b], acc_sh.at[db[j]], ssem).start(add=True)

    def s_wait(j, b):
        pltpu.make_async_copy(rr[b], acc_sh.at[db[j]], ssem).wait()


    for q in range(6):
        idx_start(q, q)
    for q in range(2):
        idx_wait(q)
        g_start(q, q % NRB)

    zmain.wait()

    @pl.when(s == NS - 1)
    def _():
        ztail.wait()

    plsc.subcore_barrier()
    for q in range(8):
        g_wait(q % NIB, q % NRB)
        s_start(q % NIB, q % NRB)
        if q >= 2:
            s_wait((q - 2) % NIB, (q - 2) % NRB)
        idx_start(q + 6, (q + 6) % NIB)
        idx_wait((q + 2) % NIB)
        g_start((q + 2) % NIB, (q + 2) % NRB)

    @pl.loop(8, SE, step=8)
    def _(k):
        for p in range(8):
            g_wait(p, p % NRB)
            s_start(p, p % NRB)
            s_wait((p - 2) % NIB, (p - 2) % NRB)

            @pl.when(k + p + 6 < NCHUNK)
            def _(p=p):
                idx_start(k + p + 6, (p + 6) % NIB)

            idx_wait((p + 2) % NIB)
            g_start((p + 2) % NIB, (p + 2) % NRB)

    for q in range(SE, NCHUNK):
        g_wait(q % NIB, q % NRB)
        s_start(q % NIB, q % NRB)
        s_wait((q - 2) % NIB, (q - 2) % NRB)
        if q + 2 < NCHUNK:
            idx_wait((q + 2) % NIB)
            g_start((q + 2) % NIB, (q + 2) % NRB)
    s_wait((NCHUNK - 2) % NIB, (NCHUNK - 2) % NRB)
    s_wait((NCHUNK - 1) % NIB, (NCHUNK - 1) % NRB)

    plsc.subcore_barrier()
    pltpu.sync_copy(acc_sh.at[pl.ds(row0, RPT)],
                    out_hbm.at[pl.ds(c * N + row0, RPT)])

    @pl.when(s == NS - 1)
    def _():
        pltpu.sync_copy(acc_sh.at[pl.ds(NS * RPT, RTAIL)],
                        out_hbm.at[pl.ds(c * N + NS * RPT, RTAIL)])


@functools.lru_cache(maxsize=1)
def _make_sc_aggregate():
    mesh = plsc.VectorSubcoreMesh(core_axis_name="c", subcore_axis_name="s")
    return pl.kernel(
        _agg_body,
            out_type=jax.ShapeDtypeStruct((NC * N, D), jnp.float32),
        mesh=mesh,
        scratch_types=(
            [pltpu.VMEM((CHUNK,), jnp.int32)] * 16
            + [pltpu.VMEM((CHUNK, D), jnp.float32)] * 4
            + [pltpu.VMEM_SHARED((N, D), jnp.float32),
               pltpu.SemaphoreType.DMA,
               pltpu.SemaphoreType.DMA,
               pltpu.SemaphoreType.DMA,
               pltpu.SemaphoreType.DMA]
        ),
    )


_VEC_SPEC = pl.BlockSpec((1, D), lambda i: (0, 0))
_MAT_SPEC = pl.BlockSpec((D, D), lambda i: (0, 0))
_BLK_SPEC = pl.BlockSpec((BLK, D), lambda i: (i, 0))


def _row(v):
    return v.reshape(1, D)


def _pre_body(eps_ref, h_ref, w1_ref, b1_ref, o_ref):
    z = h_ref[...] * (1.0 + eps_ref[0, 0])
    z = jnp.dot(z, w1_ref[...], preferred_element_type=jnp.float32,
                precision=lax.Precision.HIGHEST)
    o_ref[...] = z + b1_ref[...]


def _pre_call(h, eps, w1, b1):
    return pl.pallas_call(
        _pre_body,
        grid=(N // BLK,),
        in_specs=[pl.BlockSpec(memory_space=pltpu.SMEM),
                  _BLK_SPEC, _MAT_SPEC, _VEC_SPEC],
        out_specs=_BLK_SPEC,
        out_shape=jax.ShapeDtypeStruct((N, D), jnp.float32),
    )(eps.reshape(1, 1), h, w1, _row(b1))


def _mlp_body(p1_ref, a0_ref, a1_ref, w1_ref, g1_ref, be1_ref,
              w2_ref, b2_ref, g2_ref, be2_ref, o_ref):
    z = a0_ref[...] + a1_ref[...]
    z = jnp.dot(z, w1_ref[...], preferred_element_type=jnp.float32,
                precision=lax.Precision.HIGHEST)
    z = jnp.maximum(z + p1_ref[...], 0.0)
    z = z * (g1_ref[...] * BN_INV) + be1_ref[...]
    z = jnp.dot(z, w2_ref[...], preferred_element_type=jnp.float32,
                precision=lax.Precision.HIGHEST)
    z = (z + b2_ref[...]) * (g2_ref[...] * BN_INV) + be2_ref[...]
    o_ref[...] = jnp.maximum(z, 0.0)


def _mlp_call(p1, agg, w1, g1, be1, w2, b2, g2, be2):
    return pl.pallas_call(
        _mlp_body,
        grid=(N // BLK,),
        in_specs=[
            _BLK_SPEC,
            _BLK_SPEC,
            pl.BlockSpec((BLK, D), lambda i: (i + N // BLK, 0)),
            _MAT_SPEC, _VEC_SPEC, _VEC_SPEC,
            _MAT_SPEC, _VEC_SPEC, _VEC_SPEC, _VEC_SPEC,
        ],
        out_specs=_BLK_SPEC,
        out_shape=jax.ShapeDtypeStruct((N, D), jnp.float32),
    )(p1, agg, agg, w1, _row(g1), _row(be1),
      w2, _row(b2), _row(g2), _row(be2))


def kernel(x, edge_index, batch, params):
    del batch
    src = edge_index[0]
    dst = edge_index[1]
    zeros = jnp.zeros((N, D), jnp.float32)
    sc_aggregate = _make_sc_aggregate()
    h = x
    for i in range(L):
        agg = sc_aggregate(h, src, dst, zeros)
        p1 = _pre_call(h, params[f"eps{i}"], params[f"W1_{i}"],
                       params[f"b1_{i}"])
        h = _mlp_call(p1, agg,
                      params[f"W1_{i}"], params[f"g1_{i}"], params[f"be1_{i}"],
                      params[f"W2_{i}"], params[f"b2_{i}"],
                      params[f"g_{i}"], params[f"be_{i}"])
    return h


# speedup vs baseline: 1.2369x; 1.0087x over previous
import functools

import jax
import jax.numpy as jnp
from jax import lax
from jax.experimental import pallas as pl
from jax.experimental.pallas import tpu as pltpu
from jax.experimental.pallas import tpu_sc as plsc

N = 10000
E = 320000
D = 128
L = 5
BN_EPS = 1e-5
BN_INV = 1.0 / (1.0 + BN_EPS) ** 0.5

NC = 2
NS = 16
NW = NC * NS
EPT = E // NW
CHUNK = 80
NCHUNK = EPT // CHUNK
NIB = 8
NRB = 4
SE = 8 + 8 * ((NCHUNK - 10) // 8)
RPT = 624
RTAIL = N - NS * RPT

BLK = 2000


def _agg_body(h_hbm, src_hbm, dst_hbm, zero_hbm, out_hbm,
              sb0, sb1, sb2, sb3, sb4, sb5, sb6, sb7,
              db0, db1, db2, db3, db4, db5, db6, db7,
              r0, r1, r2, r3, acc_sh, isem, gsem, ssem, zsem):
    c = lax.axis_index("c")
    s = lax.axis_index("s")
    wid = c * NS + s
    base = wid * EPT
    row0 = s * RPT
    sb = [sb0, sb1, sb2, sb3, sb4, sb5, sb6, sb7]
    db = [db0, db1, db2, db3, db4, db5, db6, db7]
    rr = [r0, r1, r2, r3]

    zmain = pltpu.make_async_copy(zero_hbm.at[pl.ds(row0, RPT)],
                                  acc_sh.at[pl.ds(row0, RPT)], zsem)
    ztail = pltpu.make_async_copy(zero_hbm.at[pl.ds(NS * RPT, RTAIL)],
                                  acc_sh.at[pl.ds(NS * RPT, RTAIL)], zsem)
    zmain.start()

    @pl.when(s == NS - 1)
    def _():
        ztail.start()

    def idx_start(k, j):
        off = base + k * CHUNK
        pltpu.make_async_copy(src_hbm.at[pl.ds(off, CHUNK)], sb[j], isem).start()
        pltpu.make_async_copy(dst_hbm.at[pl.ds(off, CHUNK)], db[j], isem).start()

    def idx_wait(j):
        pltpu.make_async_copy(src_hbm.at[pl.ds(0, CHUNK)], sb[j], isem).wait()
        pltpu.make_async_copy(dst_hbm.at[pl.ds(0, CHUNK)], db[j], isem).wait()

    def g_start(j, b):
        pltpu.make_async_copy(h_hbm.at[sb[j]], rr[b], gsem).start()

    def g_wait(j, b):
        pltpu.make_async_copy(h_hbm.at[sb[j]], rr[b], gsem).wait()

    def s_start(j, b):
        pltpu.make_async_copy(rr[b], acc_sh.at[db[j]], ssem).start(add=True)

    def s_wait(j, b):
        pltpu.make_async_copy(rr[b], acc_sh.at[db[j]], ssem).wait()


    for q in range(6):
        idx_start(q, q)
    for q in range(2):
        idx_wait(q)
        g_start(q, q % NRB)

    zmain.wait()

    @pl.when(s == NS - 1)
    def _():
        ztail.wait()

    plsc.subcore_barrier()
    for q in range(8):
        g_wait(q % NIB, q % NRB)
        s_start(q % NIB, q % NRB)
        if q >= 2:
            s_wait((q - 2) % NIB, (q - 2) % NRB)
        idx_start(q + 6, (q + 6) % NIB)
        idx_wait((q + 2) % NIB)
        g_start((q + 2) % NIB, (q + 2) % NRB)

    @pl.loop(8, SE, step=8)
    def _(k):
        for p in range(8):
            g_wait(p, p % NRB)
            s_start(p, p % NRB)
            s_wait((p - 2) % NIB, (p - 2) % NRB)

            @pl.when(k + p + 6 < NCHUNK)
            def _(p=p):
                idx_start(k + p + 6, (p + 6) % NIB)

            idx_wait((p + 2) % NIB)
            g_start((p + 2) % NIB, (p + 2) % NRB)

    for q in range(SE, NCHUNK):
        g_wait(q % NIB, q % NRB)
        s_start(q % NIB, q % NRB)
        s_wait((q - 2) % NIB, (q - 2) % NRB)
        if q + 2 < NCHUNK:
            idx_wait((q + 2) % NIB)
            g_start((q + 2) % NIB, (q + 2) % NRB)
    s_wait((NCHUNK - 2) % NIB, (NCHUNK - 2) % NRB)
    s_wait((NCHUNK - 1) % NIB, (NCHUNK - 1) % NRB)

    plsc.subcore_barrier()
    pltpu.sync_copy(acc_sh.at[pl.ds(row0, RPT)],
                    out_hbm.at[pl.ds(c * N + row0, RPT)])

    @pl.when(s == NS - 1)
    def _():
        pltpu.sync_copy(acc_sh.at[pl.ds(NS * RPT, RTAIL)],
                        out_hbm.at[pl.ds(c * N + NS * RPT, RTAIL)])


@functools.lru_cache(maxsize=1)
def _make_sc_aggregate():
    mesh = plsc.VectorSubcoreMesh(core_axis_name="c", subcore_axis_name="s")
    return pl.kernel(
        _agg_body,
            out_type=jax.ShapeDtypeStruct((NC * N, D), jnp.float32),
        mesh=mesh,
        scratch_types=(
            [pltpu.VMEM((CHUNK,), jnp.int32)] * 16
            + [pltpu.VMEM((CHUNK, D), jnp.float32)] * 4
            + [pltpu.VMEM_SHARED((N, D), jnp.float32),
               pltpu.SemaphoreType.DMA,
               pltpu.SemaphoreType.DMA,
               pltpu.SemaphoreType.DMA,
               pltpu.SemaphoreType.DMA]
        ),
    )


_VEC_SPEC = pl.BlockSpec((1, D), lambda i: (0, 0))
_MAT_SPEC = pl.BlockSpec((D, D), lambda i: (0, 0))
_BLK_SPEC = pl.BlockSpec((BLK, D), lambda i: (i, 0))


def _row(v):
    return v.reshape(1, D)


def _mlp_body(eps_ref, h_ref, a0_ref, a1_ref, w1_ref, b1_ref, g1_ref, be1_ref,
              w2_ref, b2_ref, g2_ref, be2_ref, o_ref):
    z = h_ref[...] * (1.0 + eps_ref[0, 0]) + a0_ref[...] + a1_ref[...]
    z = jnp.dot(z, w1_ref[...], preferred_element_type=jnp.float32,
                precision=lax.Precision.HIGHEST)
    z = jnp.maximum(z + b1_ref[...], 0.0)
    z = z * (g1_ref[...] * BN_INV) + be1_ref[...]
    z = jnp.dot(z, w2_ref[...], preferred_element_type=jnp.float32,
                precision=lax.Precision.HIGHEST)
    z = (z + b2_ref[...]) * (g2_ref[...] * BN_INV) + be2_ref[...]
    o_ref[...] = jnp.maximum(z, 0.0)


def _mlp_call(h, agg, eps, w1, b1, g1, be1, w2, b2, g2, be2):
    return pl.pallas_call(
        _mlp_body,
        grid=(N // BLK,),
        in_specs=[
            pl.BlockSpec(memory_space=pltpu.SMEM),
            _BLK_SPEC,
            _BLK_SPEC,
            pl.BlockSpec((BLK, D), lambda i: (i + N // BLK, 0)),
            _MAT_SPEC, _VEC_SPEC, _VEC_SPEC, _VEC_SPEC,
            _MAT_SPEC, _VEC_SPEC, _VEC_SPEC, _VEC_SPEC,
        ],
        out_specs=_BLK_SPEC,
        out_shape=jax.ShapeDtypeStruct((N, D), jnp.float32),
    )(eps.reshape(1, 1), h, agg, agg, w1, _row(b1), _row(g1), _row(be1),
      w2, _row(b2), _row(g2), _row(be2))


def kernel(x, edge_index, batch, params):
    del batch
    src = edge_index[0]
    dst = edge_index[1]
    zeros = jnp.zeros((N, D), jnp.float32)
    sc_aggregate = _make_sc_aggregate()
    h = x
    for i in range(L):
        agg = sc_aggregate(h, src, dst, zeros)
        h = _mlp_call(h, agg, params[f"eps{i}"],
                      params[f"W1_{i}"], params[f"b1_{i}"],
                      params[f"g1_{i}"], params[f"be1_{i}"],
                      params[f"W2_{i}"], params[f"b2_{i}"],
                      params[f"g_{i}"], params[f"be_{i}"])
    return h
